# Initial kernel scaffold; baseline (speedup 1.0000x reference)
import jax
import jax.numpy as jnp
from jax.experimental import pallas as pl


def kernel(node_embeddings, edge_index, edge_embeddings, W1, b1, W2, b2, W3, b3, W4, b4, gamma, beta):
    raise NotImplementedError("write your pallas kernel here")


# speedup vs baseline: 3.2309x; 3.2309x over previous
import jax
import jax.numpy as jnp
from jax import lax
from jax.experimental import pallas as pl
from jax.experimental.pallas import tpu as pltpu
from jax.experimental.pallas import tpu_sc as plsc

N = 10000
E = 320000
D = 128

NC = 2
NS = 16
NW = NC * NS

CK = 128
NCHUNK = E // CK
BASE_CH = NCHUNK // NW
REM = NCHUNK % NW
NP = 10112
ROWS_PER = NP // NS
ZR = 8


def _gelu(x):
    return 0.5 * x * (1.0 + lax.erf(x * 0.7071067811865476))


def _worker_span():
    w = lax.axis_index("c") * NS + lax.axis_index("s")
    start = w * BASE_CH + jnp.minimum(w, REM)
    n = BASE_CH + jnp.where(w < REM, 1, 0)
    return start, n


def _gather_body(table, srcidx, dstidx, out, deg0, deg1,
                 sh_deg, idx_v, didx_v, rows_v, ones_v, zd, sem):
    cid = lax.axis_index("c")
    sid = lax.axis_index("s")
    start, n = _worker_span()

    zero16 = jnp.zeros((16,), jnp.float32)
    one16 = jnp.ones((16,), jnp.float32)

    def fill_zd(i, carry):
        def fl(j, c2):
            zd[i, pl.ds(j * 16, 16)] = zero16
            return c2
        return lax.fori_loop(0, D // 16, fl, carry)

    lax.fori_loop(0, ZR, fill_zd, 0)

    def fill_ones(i, carry):
        def fl(j, c2):
            ones_v[i, pl.ds(j * 16, 16)] = one16
            return c2
        return lax.fori_loop(0, D // 16, fl, carry)

    lax.fori_loop(0, CK, fill_ones, 0)

    def zero_shared(r, carry):
        pltpu.sync_copy(zd, sh_deg.at[pl.ds(sid * ROWS_PER + r * ZR, ZR)])
        return carry

    lax.fori_loop(0, ROWS_PER // ZR, zero_shared, 0)
    plsc.subcore_barrier()

    def step(t, carry):
        off = (start + t) * CK
        pltpu.sync_copy(srcidx.at[pl.ds(off, CK)], idx_v)
        pltpu.async_copy(table.at[idx_v], rows_v, sem).wait()
        pltpu.sync_copy(rows_v, out.at[pl.ds(off, CK)])
        pltpu.sync_copy(dstidx.at[pl.ds(off, CK)], didx_v)
        pltpu.sync_copy(ones_v, sh_deg.at[didx_v], add=True)
        return carry

    lax.fori_loop(0, n, step, 0)
    plsc.subcore_barrier()

    row = sid * ROWS_PER

    @pl.when(cid == 0)
    def _():
        pltpu.sync_copy(sh_deg.at[pl.ds(row, ROWS_PER)],
                        deg0.at[pl.ds(row, ROWS_PER)])

    @pl.when(cid == 1)
    def _():
        pltpu.sync_copy(sh_deg.at[pl.ds(row, ROWS_PER)],
                        deg1.at[pl.ds(row, ROWS_PER)])


def _sc_gather_deg(node_embeddings, src, dst):
    mesh = plsc.VectorSubcoreMesh(core_axis_name="c", subcore_axis_name="s")
    f = pl.kernel(
        _gather_body,
        out_type=(
            jax.ShapeDtypeStruct((E, D), jnp.float32),
            jax.ShapeDtypeStruct((NP, D), jnp.float32),
            jax.ShapeDtypeStruct((NP, D), jnp.float32),
        ),
        mesh=mesh,
        scratch_types=[
            pltpu.VMEM_SHARED((NP, D), jnp.float32),
            pltpu.VMEM((CK,), jnp.int32),
            pltpu.VMEM((CK,), jnp.int32),
            pltpu.VMEM((CK, D), jnp.float32),
            pltpu.VMEM((CK, D), jnp.float32),
            pltpu.VMEM((ZR, D), jnp.float32),
            pltpu.SemaphoreType.DMA,
        ],
    )
    return f(node_embeddings, src, dst)


def _scatter_body(msg, dstidx, agg0, agg1,
                  sh_agg, idx_v, rows_v, zb, sem):
    cid = lax.axis_index("c")
    sid = lax.axis_index("s")
    start, n = _worker_span()

    zero16 = jnp.zeros((16,), jnp.float32)

    def fill_zb(i, carry):
        def fill_lane(j, c2):
            zb[i, pl.ds(j * 16, 16)] = zero16
            return c2
        return lax.fori_loop(0, D // 16, fill_lane, carry)

    lax.fori_loop(0, ZR, fill_zb, 0)

    def zero_shared(r, carry):
        pltpu.sync_copy(zb, sh_agg.at[pl.ds(sid * ROWS_PER + r * ZR, ZR)])
        return carry

    lax.fori_loop(0, ROWS_PER // ZR, zero_shared, 0)
    plsc.subcore_barrier()

    def step(t, carry):
        off = (start + t) * CK
        pltpu.sync_copy(dstidx.at[pl.ds(off, CK)], idx_v)
        pltpu.sync_copy(msg.at[pl.ds(off, CK)], rows_v)
        pltpu.sync_copy(rows_v, sh_agg.at[idx_v], add=True)
        return carry

    lax.fori_loop(0, n, step, 0)
    plsc.subcore_barrier()

    row = sid * ROWS_PER

    @pl.when(cid == 0)
    def _():
        pltpu.sync_copy(sh_agg.at[pl.ds(row, ROWS_PER)],
                        agg0.at[pl.ds(row, ROWS_PER)])

    @pl.when(cid == 1)
    def _():
        pltpu.sync_copy(sh_agg.at[pl.ds(row, ROWS_PER)],
                        agg1.at[pl.ds(row, ROWS_PER)])


def _sc_scatter(messages, dst):
    mesh = plsc.VectorSubcoreMesh(core_axis_name="c", subcore_axis_name="s")
    f = pl.kernel(
        _scatter_body,
        out_type=(
            jax.ShapeDtypeStruct((NP, D), jnp.float32),
            jax.ShapeDtypeStruct((NP, D), jnp.float32),
        ),
        mesh=mesh,
        scratch_types=[
            pltpu.VMEM_SHARED((NP, D), jnp.float32),
            pltpu.VMEM((CK,), jnp.int32),
            pltpu.VMEM((CK, D), jnp.float32),
            pltpu.VMEM((ZR, D), jnp.float32),
            pltpu.SemaphoreType.DMA,
        ],
    )
    return f(messages, dst)


BE = 2000


def _edge_body(g_ref, e_ref, w1_ref, b1_ref, w2_ref, b2_ref, o_ref):
    x = g_ref[...] + e_ref[...]
    h = jnp.dot(x, w1_ref[...], preferred_element_type=jnp.float32) + b1_ref[...]
    h = _gelu(h)
    o_ref[...] = (jnp.dot(h, w2_ref[...], preferred_element_type=jnp.float32)
                  + b2_ref[...])


def _tc_edge_mlp(gathered, edge_embeddings, W1, b1, W2, b2):
    grid = (E // BE,)
    return pl.pallas_call(
        _edge_body,
        grid=grid,
        in_specs=[
            pl.BlockSpec((BE, D), lambda i: (i, 0)),
            pl.BlockSpec((BE, D), lambda i: (i, 0)),
            pl.BlockSpec((D, D), lambda i: (0, 0)),
            pl.BlockSpec((1, D), lambda i: (0, 0)),
            pl.BlockSpec((D, D), lambda i: (0, 0)),
            pl.BlockSpec((1, D), lambda i: (0, 0)),
        ],
        out_specs=pl.BlockSpec((BE, D), lambda i: (i, 0)),
        out_shape=jax.ShapeDtypeStruct((E, D), jnp.float32),
    )(gathered, edge_embeddings, W1, b1.reshape(1, D), W2, b2.reshape(1, D))


BN = 2000


def _node_body(node_ref, a0_ref, a1_ref, d0_ref, d1_ref,
               w3_ref, b3_ref, w4_ref, b4_ref, g_ref, be_ref, o_ref):
    p = a0_ref[...] + a1_ref[...]
    deg = d0_ref[...] + d1_ref[...]
    a = p / jnp.maximum(deg, 1.0)
    t = _gelu(jnp.dot(a, w3_ref[...], preferred_element_type=jnp.float32)
              + b3_ref[...])
    u = jnp.dot(t, w4_ref[...], preferred_element_type=jnp.float32) + b4_ref[...]
    y = node_ref[...] + u
    mean = jnp.mean(y, axis=-1, keepdims=True)
    var = jnp.mean((y - mean) * (y - mean), axis=-1, keepdims=True)
    o_ref[...] = ((y - mean) * lax.rsqrt(var + 1e-5)) * g_ref[...] + be_ref[...]


def _tc_node_mlp(node_embeddings, agg0, agg1, deg0, deg1, W3, b3, W4, b4,
                 gamma, beta):
    grid = (N // BN,)
    return pl.pallas_call(
        _node_body,
        grid=grid,
        in_specs=[
            pl.BlockSpec((BN, D), lambda i: (i, 0)),
            pl.BlockSpec((BN, D), lambda i: (i, 0)),
            pl.BlockSpec((BN, D), lambda i: (i, 0)),
            pl.BlockSpec((BN, D), lambda i: (i, 0)),
            pl.BlockSpec((BN, D), lambda i: (i, 0)),
            pl.BlockSpec((D, 2 * D), lambda i: (0, 0)),
            pl.BlockSpec((1, 2 * D), lambda i: (0, 0)),
            pl.BlockSpec((2 * D, D), lambda i: (0, 0)),
            pl.BlockSpec((1, D), lambda i: (0, 0)),
            pl.BlockSpec((1, D), lambda i: (0, 0)),
            pl.BlockSpec((1, D), lambda i: (0, 0)),
        ],
        out_specs=pl.BlockSpec((BN, D), lambda i: (i, 0)),
        out_shape=jax.ShapeDtypeStruct((N, D), jnp.float32),
    )(node_embeddings, agg0, agg1, deg0, deg1,
      W3, b3.reshape(1, 2 * D), W4, b4.reshape(1, D),
      gamma.reshape(1, D), beta.reshape(1, D))


def kernel(node_embeddings, edge_index, edge_embeddings,
           W1, b1, W2, b2, W3, b3, W4, b4, gamma, beta):
    src = edge_index[0]
    dst = edge_index[1]
    gathered, deg0, deg1 = _sc_gather_deg(node_embeddings, src, dst)
    messages = _tc_edge_mlp(gathered, edge_embeddings, W1, b1, W2, b2)
    agg0, agg1 = _sc_scatter(messages, dst)
    return _tc_node_mlp(node_embeddings, agg0, agg1, deg0, deg1,
                        W3, b3, W4, b4, gamma, beta)


# speedup vs baseline: 4.1019x; 1.2696x over previous
import jax
import jax.numpy as jnp
from jax import lax
from jax.experimental import pallas as pl
from jax.experimental.pallas import tpu as pltpu
from jax.experimental.pallas import tpu_sc as plsc

N = 10000
E = 320000
D = 128

NC = 2
NS = 16
NW = NC * NS

CK = 256
NCHUNK = E // CK
BASE_CH = NCHUNK // NW
REM = NCHUNK % NW
NP = 10112
ROWS_PER = NP // NS
ZR = 8
NPD = 10240
SPD = NPD // NS


def _gelu(x):
    return 0.5 * x * (1.0 + lax.erf(x * 0.7071067811865476))


def _worker_span():
    w = lax.axis_index("c") * NS + lax.axis_index("s")
    start = w * BASE_CH + jnp.minimum(w, REM)
    n = BASE_CH + jnp.where(w < REM, 1, 0)
    return start, n


def _gather_body(table, srcidx, dstidx, out, deg0, deg1,
                 sh_deg, idx_v, didx_v, rows_v, ones_v, iota_v, zeros_v,
                 buf_v, sem):
    cid = lax.axis_index("c")
    sid = lax.axis_index("s")
    start, n = _worker_span()

    zero16 = jnp.zeros((16,), jnp.float32)
    one16 = jnp.ones((16,), jnp.float32)

    def fill_ones(i, carry):
        ones_v[pl.ds(i * 16, 16)] = one16
        return carry

    lax.fori_loop(0, CK // 16, fill_ones, 0)

    def fill_iota(i, carry):
        zeros_v[pl.ds(i * 16, 16)] = zero16
        iota_v[pl.ds(i * 16, 16)] = (lax.iota(jnp.int32, 16)
                                     + (sid * SPD + i * 16))
        return carry

    lax.fori_loop(0, SPD // 16, fill_iota, 0)

    pltpu.sync_copy(zeros_v, sh_deg.at[iota_v])
    plsc.subcore_barrier()

    def step(t, carry):
        off = (start + t) * CK
        pltpu.sync_copy(srcidx.at[pl.ds(off, CK)], idx_v)
        pltpu.async_copy(table.at[idx_v], rows_v, sem).wait()
        pltpu.sync_copy(rows_v, out.at[pl.ds(off, CK)])
        pltpu.sync_copy(dstidx.at[pl.ds(off, CK)], didx_v)
        pltpu.sync_copy(ones_v, sh_deg.at[didx_v], add=True)
        return carry

    lax.fori_loop(0, n, step, 0)
    plsc.subcore_barrier()

    pltpu.sync_copy(sh_deg.at[iota_v], buf_v)

    @pl.when(cid == 0)
    def _():
        pltpu.sync_copy(buf_v, deg0.at[pl.ds(sid * SPD, SPD)])

    @pl.when(cid == 1)
    def _():
        pltpu.sync_copy(buf_v, deg1.at[pl.ds(sid * SPD, SPD)])


def _sc_gather_deg(node_embeddings, src, dst):
    mesh = plsc.VectorSubcoreMesh(core_axis_name="c", subcore_axis_name="s")
    f = pl.kernel(
        _gather_body,
        out_type=(
            jax.ShapeDtypeStruct((E, D), jnp.float32),
            jax.ShapeDtypeStruct((NPD,), jnp.float32),
            jax.ShapeDtypeStruct((NPD,), jnp.float32),
        ),
        mesh=mesh,
        scratch_types=[
            pltpu.VMEM_SHARED((NPD,), jnp.float32),
            pltpu.VMEM((CK,), jnp.int32),
            pltpu.VMEM((CK,), jnp.int32),
            pltpu.VMEM((CK, D), jnp.float32),
            pltpu.VMEM((CK,), jnp.float32),
            pltpu.VMEM((SPD,), jnp.int32),
            pltpu.VMEM((SPD,), jnp.float32),
            pltpu.VMEM((SPD,), jnp.float32),
            pltpu.SemaphoreType.DMA,
        ],
    )
    return f(node_embeddings, src, dst)


def _scatter_body(msg, dstidx, agg0, agg1,
                  sh_agg, idx_v, rows_v, zb, sem):
    cid = lax.axis_index("c")
    sid = lax.axis_index("s")
    start, n = _worker_span()

    zero16 = jnp.zeros((16,), jnp.float32)

    def fill_zb(i, carry):
        def fill_lane(j, c2):
            zb[i, pl.ds(j * 16, 16)] = zero16
            return c2
        return lax.fori_loop(0, D // 16, fill_lane, carry)

    lax.fori_loop(0, ZR, fill_zb, 0)

    def zero_shared(r, carry):
        pltpu.sync_copy(zb, sh_agg.at[pl.ds(sid * ROWS_PER + r * ZR, ZR)])
        return carry

    lax.fori_loop(0, ROWS_PER // ZR, zero_shared, 0)
    plsc.subcore_barrier()

    def step(t, carry):
        off = (start + t) * CK
        pltpu.sync_copy(dstidx.at[pl.ds(off, CK)], idx_v)
        pltpu.sync_copy(msg.at[pl.ds(off, CK)], rows_v)
        pltpu.sync_copy(rows_v, sh_agg.at[idx_v], add=True)
        return carry

    lax.fori_loop(0, n, step, 0)
    plsc.subcore_barrier()

    row = sid * ROWS_PER

    @pl.when(cid == 0)
    def _():
        pltpu.sync_copy(sh_agg.at[pl.ds(row, ROWS_PER)],
                        agg0.at[pl.ds(row, ROWS_PER)])

    @pl.when(cid == 1)
    def _():
        pltpu.sync_copy(sh_agg.at[pl.ds(row, ROWS_PER)],
                        agg1.at[pl.ds(row, ROWS_PER)])


def _sc_scatter(messages, dst):
    mesh = plsc.VectorSubcoreMesh(core_axis_name="c", subcore_axis_name="s")
    f = pl.kernel(
        _scatter_body,
        out_type=(
            jax.ShapeDtypeStruct((NP, D), jnp.float32),
            jax.ShapeDtypeStruct((NP, D), jnp.float32),
        ),
        mesh=mesh,
        scratch_types=[
            pltpu.VMEM_SHARED((NP, D), jnp.float32),
            pltpu.VMEM((CK,), jnp.int32),
            pltpu.VMEM((CK, D), jnp.float32),
            pltpu.VMEM((ZR, D), jnp.float32),
            pltpu.SemaphoreType.DMA,
        ],
    )
    return f(messages, dst)


BE = 2000


def _edge_body(g_ref, e_ref, w1_ref, b1_ref, w2_ref, b2_ref, o_ref):
    x = g_ref[...] + e_ref[...]
    h = jnp.dot(x, w1_ref[...], preferred_element_type=jnp.float32) + b1_ref[...]
    h = _gelu(h)
    o_ref[...] = (jnp.dot(h, w2_ref[...], preferred_element_type=jnp.float32)
                  + b2_ref[...])


def _tc_edge_mlp(gathered, edge_embeddings, W1, b1, W2, b2):
    grid = (E // BE,)
    return pl.pallas_call(
        _edge_body,
        grid=grid,
        in_specs=[
            pl.BlockSpec((BE, D), lambda i: (i, 0)),
            pl.BlockSpec((BE, D), lambda i: (i, 0)),
            pl.BlockSpec((D, D), lambda i: (0, 0)),
            pl.BlockSpec((1, D), lambda i: (0, 0)),
            pl.BlockSpec((D, D), lambda i: (0, 0)),
            pl.BlockSpec((1, D), lambda i: (0, 0)),
        ],
        out_specs=pl.BlockSpec((BE, D), lambda i: (i, 0)),
        out_shape=jax.ShapeDtypeStruct((E, D), jnp.float32),
    )(gathered, edge_embeddings, W1, b1.reshape(1, D), W2, b2.reshape(1, D))


BN = 2000


def _node_body(node_ref, a0_ref, a1_ref, d0_ref, d1_ref,
               w3_ref, b3_ref, w4_ref, b4_ref, g_ref, be_ref, o_ref):
    p = a0_ref[...] + a1_ref[...]
    deg = d0_ref[...] + d1_ref[...]
    a = p / jnp.maximum(deg, 1.0)
    t = _gelu(jnp.dot(a, w3_ref[...], preferred_element_type=jnp.float32)
              + b3_ref[...])
    u = jnp.dot(t, w4_ref[...], preferred_element_type=jnp.float32) + b4_ref[...]
    y = node_ref[...] + u
    mean = jnp.mean(y, axis=-1, keepdims=True)
    var = jnp.mean((y - mean) * (y - mean), axis=-1, keepdims=True)
    o_ref[...] = ((y - mean) * lax.rsqrt(var + 1e-5)) * g_ref[...] + be_ref[...]


def _tc_node_mlp(node_embeddings, agg0, agg1, deg0, deg1, W3, b3, W4, b4,
                 gamma, beta):
    grid = (N // BN,)
    return pl.pallas_call(
        _node_body,
        grid=grid,
        in_specs=[
            pl.BlockSpec((BN, D), lambda i: (i, 0)),
            pl.BlockSpec((BN, D), lambda i: (i, 0)),
            pl.BlockSpec((BN, D), lambda i: (i, 0)),
            pl.BlockSpec((BN, 1), lambda i: (i, 0)),
            pl.BlockSpec((BN, 1), lambda i: (i, 0)),
            pl.BlockSpec((D, 2 * D), lambda i: (0, 0)),
            pl.BlockSpec((1, 2 * D), lambda i: (0, 0)),
            pl.BlockSpec((2 * D, D), lambda i: (0, 0)),
            pl.BlockSpec((1, D), lambda i: (0, 0)),
            pl.BlockSpec((1, D), lambda i: (0, 0)),
            pl.BlockSpec((1, D), lambda i: (0, 0)),
        ],
        out_specs=pl.BlockSpec((BN, D), lambda i: (i, 0)),
        out_shape=jax.ShapeDtypeStruct((N, D), jnp.float32),
    )(node_embeddings, agg0, agg1, deg0, deg1,
      W3, b3.reshape(1, 2 * D), W4, b4.reshape(1, D),
      gamma.reshape(1, D), beta.reshape(1, D))


def kernel(node_embeddings, edge_index, edge_embeddings,
           W1, b1, W2, b2, W3, b3, W4, b4, gamma, beta):
    src = edge_index[0]
    dst = edge_index[1]
    gathered, deg0, deg1 = _sc_gather_deg(node_embeddings, src, dst)
    messages = _tc_edge_mlp(gathered, edge_embeddings, W1, b1, W2, b2)
    agg0, agg1 = _sc_scatter(messages, dst)
    d0 = deg0[:N].reshape(N, 1)
    d1 = deg1[:N].reshape(N, 1)
    return _tc_node_mlp(node_embeddings, agg0, agg1, d0, d1,
                        W3, b3, W4, b4, gamma, beta)


# speedup vs baseline: 5.0192x; 1.2236x over previous
import jax
import jax.numpy as jnp
from jax import lax
from jax.experimental import pallas as pl
from jax.experimental.pallas import tpu as pltpu
from jax.experimental.pallas import tpu_sc as plsc

N = 10000
E = 320000
D = 128

NC = 2
NS = 16
NW = NC * NS

H = 2
E2 = E // H
CK = 256
NCHUNK = E2 // CK
BASE_CH = NCHUNK // NW
REM = NCHUNK % NW
NP = 10112
ROWS_PER = NP // NS
ZR = 8
NPD = 10240
SPD = NPD // NS


def _gelu(x):
    return 0.5 * x * (1.0 + lax.erf(x * 0.7071067811865476))


def _worker_span():
    w = lax.axis_index("c") * NS + lax.axis_index("s")
    start = w * BASE_CH + jnp.minimum(w, REM)
    n = BASE_CH + jnp.where(w < REM, 1, 0)
    return start, n


def _make_gather(hoff):
    def body(table, srcidx, dstidx, out, deg0, deg1,
             sh_deg, idx_v, didx_v, rows_v, ones_v, iota_v, zeros_v,
             buf_v, sem):
        cid = lax.axis_index("c")
        sid = lax.axis_index("s")
        start, n = _worker_span()

        zero16 = jnp.zeros((16,), jnp.float32)
        one16 = jnp.ones((16,), jnp.float32)

        def fill_ones(i, carry):
            ones_v[pl.ds(i * 16, 16)] = one16
            return carry

        lax.fori_loop(0, CK // 16, fill_ones, 0)

        def fill_iota(i, carry):
            zeros_v[pl.ds(i * 16, 16)] = zero16
            iota_v[pl.ds(i * 16, 16)] = (lax.iota(jnp.int32, 16)
                                         + (sid * SPD + i * 16))
            return carry

        lax.fori_loop(0, SPD // 16, fill_iota, 0)

        pltpu.sync_copy(zeros_v, sh_deg.at[iota_v])
        plsc.subcore_barrier()

        def step(t, carry):
            loff = (start + t) * CK
            goff = hoff + loff
            pltpu.sync_copy(srcidx.at[pl.ds(goff, CK)], idx_v)
            pltpu.async_copy(table.at[idx_v], rows_v, sem).wait()
            pltpu.sync_copy(rows_v, out.at[pl.ds(loff, CK)])
            pltpu.sync_copy(dstidx.at[pl.ds(goff, CK)], didx_v)
            pltpu.sync_copy(ones_v, sh_deg.at[didx_v], add=True)
            return carry

        lax.fori_loop(0, n, step, 0)
        plsc.subcore_barrier()

        pltpu.sync_copy(sh_deg.at[iota_v], buf_v)

        @pl.when(cid == 0)
        def _():
            pltpu.sync_copy(buf_v, deg0.at[pl.ds(sid * SPD, SPD)])

        @pl.when(cid == 1)
        def _():
            pltpu.sync_copy(buf_v, deg1.at[pl.ds(sid * SPD, SPD)])

    return body


def _sc_gather_deg(node_embeddings, src, dst, hoff):
    mesh = plsc.VectorSubcoreMesh(core_axis_name="c", subcore_axis_name="s")
    f = pl.kernel(
        _make_gather(hoff),
        out_type=(
            jax.ShapeDtypeStruct((E2, D), jnp.float32),
            jax.ShapeDtypeStruct((NPD,), jnp.float32),
            jax.ShapeDtypeStruct((NPD,), jnp.float32),
        ),
        mesh=mesh,
        scratch_types=[
            pltpu.VMEM_SHARED((NPD,), jnp.float32),
            pltpu.VMEM((CK,), jnp.int32),
            pltpu.VMEM((CK,), jnp.int32),
            pltpu.VMEM((CK, D), jnp.float32),
            pltpu.VMEM((CK,), jnp.float32),
            pltpu.VMEM((SPD,), jnp.int32),
            pltpu.VMEM((SPD,), jnp.float32),
            pltpu.VMEM((SPD,), jnp.float32),
            pltpu.SemaphoreType.DMA,
        ],
    )
    return f(node_embeddings, src, dst)


def _make_scatter(hoff):
    def body(msg, dstidx, agg0, agg1, sh_agg, idx_v, rows_v, zb, sem):
        cid = lax.axis_index("c")
        sid = lax.axis_index("s")
        start, n = _worker_span()

        zero16 = jnp.zeros((16,), jnp.float32)

        def fill_zb(i, carry):
            def fill_lane(j, c2):
                zb[i, pl.ds(j * 16, 16)] = zero16
                return c2
            return lax.fori_loop(0, D // 16, fill_lane, carry)

        lax.fori_loop(0, ZR, fill_zb, 0)

        def zero_shared(r, carry):
            pltpu.sync_copy(zb, sh_agg.at[pl.ds(sid * ROWS_PER + r * ZR, ZR)])
            return carry

        lax.fori_loop(0, ROWS_PER // ZR, zero_shared, 0)
        plsc.subcore_barrier()

        def step(t, carry):
            loff = (start + t) * CK
            pltpu.sync_copy(dstidx.at[pl.ds(hoff + loff, CK)], idx_v)
            pltpu.sync_copy(msg.at[pl.ds(loff, CK)], rows_v)
            pltpu.sync_copy(rows_v, sh_agg.at[idx_v], add=True)
            return carry

        lax.fori_loop(0, n, step, 0)
        plsc.subcore_barrier()

        row = sid * ROWS_PER

        @pl.when(cid == 0)
        def _():
            pltpu.sync_copy(sh_agg.at[pl.ds(row, ROWS_PER)],
                            agg0.at[pl.ds(row, ROWS_PER)])

        @pl.when(cid == 1)
        def _():
            pltpu.sync_copy(sh_agg.at[pl.ds(row, ROWS_PER)],
                            agg1.at[pl.ds(row, ROWS_PER)])

    return body


def _sc_scatter(messages, dst, hoff):
    mesh = plsc.VectorSubcoreMesh(core_axis_name="c", subcore_axis_name="s")
    f = pl.kernel(
        _make_scatter(hoff),
        out_type=(
            jax.ShapeDtypeStruct((NP, D), jnp.float32),
            jax.ShapeDtypeStruct((NP, D), jnp.float32),
        ),
        mesh=mesh,
        scratch_types=[
            pltpu.VMEM_SHARED((NP, D), jnp.float32),
            pltpu.VMEM((CK,), jnp.int32),
            pltpu.VMEM((CK, D), jnp.float32),
            pltpu.VMEM((ZR, D), jnp.float32),
            pltpu.SemaphoreType.DMA,
        ],
    )
    return f(messages, dst)


BE = 2000


def _edge_body(g_ref, e_ref, w1_ref, b1_ref, w2_ref, b2_ref, o_ref):
    x = g_ref[...] + e_ref[...]
    h = jnp.dot(x, w1_ref[...], preferred_element_type=jnp.float32) + b1_ref[...]
    h = _gelu(h)
    o_ref[...] = (jnp.dot(h, w2_ref[...], preferred_element_type=jnp.float32)
                  + b2_ref[...])


def _tc_edge_mlp(gathered, edge_embeddings, W1, b1, W2, b2, eblk):
    grid = (E2 // BE,)
    return pl.pallas_call(
        _edge_body,
        grid=grid,
        in_specs=[
            pl.BlockSpec((BE, D), lambda i: (i, 0)),
            pl.BlockSpec((BE, D), lambda i, _o=eblk: (i + _o, 0)),
            pl.BlockSpec((D, D), lambda i: (0, 0)),
            pl.BlockSpec((1, D), lambda i: (0, 0)),
            pl.BlockSpec((D, D), lambda i: (0, 0)),
            pl.BlockSpec((1, D), lambda i: (0, 0)),
        ],
        out_specs=pl.BlockSpec((BE, D), lambda i: (i, 0)),
        out_shape=jax.ShapeDtypeStruct((E2, D), jnp.float32),
    )(gathered, edge_embeddings, W1, b1.reshape(1, D), W2, b2.reshape(1, D))


BN = 2000


def _node_body(node_ref, a0_ref, a1_ref, a2_ref, a3_ref,
               d0_ref, d1_ref, d2_ref, d3_ref,
               w3_ref, b3_ref, w4_ref, b4_ref, g_ref, be_ref, o_ref):
    p = (a0_ref[...] + a1_ref[...]) + (a2_ref[...] + a3_ref[...])
    deg = (d0_ref[...] + d1_ref[...]) + (d2_ref[...] + d3_ref[...])
    a = p / jnp.maximum(deg, 1.0)
    t = _gelu(jnp.dot(a, w3_ref[...], preferred_element_type=jnp.float32)
              + b3_ref[...])
    u = jnp.dot(t, w4_ref[...], preferred_element_type=jnp.float32) + b4_ref[...]
    y = node_ref[...] + u
    mean = jnp.mean(y, axis=-1, keepdims=True)
    var = jnp.mean((y - mean) * (y - mean), axis=-1, keepdims=True)
    o_ref[...] = ((y - mean) * lax.rsqrt(var + 1e-5)) * g_ref[...] + be_ref[...]


def _tc_node_mlp(node_embeddings, aggs, degs, W3, b3, W4, b4, gamma, beta):
    grid = (N // BN,)
    agg_spec = pl.BlockSpec((BN, D), lambda i: (i, 0))
    deg_spec = pl.BlockSpec((BN, 1), lambda i: (i, 0))
    return pl.pallas_call(
        _node_body,
        grid=grid,
        in_specs=[
            pl.BlockSpec((BN, D), lambda i: (i, 0)),
            agg_spec, agg_spec, agg_spec, agg_spec,
            deg_spec, deg_spec, deg_spec, deg_spec,
            pl.BlockSpec((D, 2 * D), lambda i: (0, 0)),
            pl.BlockSpec((1, 2 * D), lambda i: (0, 0)),
            pl.BlockSpec((2 * D, D), lambda i: (0, 0)),
            pl.BlockSpec((1, D), lambda i: (0, 0)),
            pl.BlockSpec((1, D), lambda i: (0, 0)),
            pl.BlockSpec((1, D), lambda i: (0, 0)),
        ],
        out_specs=pl.BlockSpec((BN, D), lambda i: (i, 0)),
        out_shape=jax.ShapeDtypeStruct((N, D), jnp.float32),
    )(node_embeddings, *aggs, *degs,
      W3, b3.reshape(1, 2 * D), W4, b4.reshape(1, D),
      gamma.reshape(1, D), beta.reshape(1, D))


def kernel(node_embeddings, edge_index, edge_embeddings,
           W1, b1, W2, b2, W3, b3, W4, b4, gamma, beta):
    src = edge_index[0]
    dst = edge_index[1]

    g0, deg00, deg01 = _sc_gather_deg(node_embeddings, src, dst, 0)
    g1, deg10, deg11 = _sc_gather_deg(node_embeddings, src, dst, E2)

    m0 = _tc_edge_mlp(g0, edge_embeddings, W1, b1, W2, b2, 0)
    m1 = _tc_edge_mlp(g1, edge_embeddings, W1, b1, W2, b2, E2 // BE)

    a00, a01 = _sc_scatter(m0, dst, 0)
    a10, a11 = _sc_scatter(m1, dst, E2)

    degs = [d[:N].reshape(N, 1) for d in (deg00, deg01, deg10, deg11)]
    return _tc_node_mlp(node_embeddings, (a00, a01, a10, a11), degs,
                        W3, b3, W4, b4, gamma, beta)


# speedup vs baseline: 5.3350x; 1.0629x over previous
import jax
import jax.numpy as jnp
from jax import lax
from jax.experimental import pallas as pl
from jax.experimental.pallas import tpu as pltpu
from jax.experimental.pallas import tpu_sc as plsc

N = 10000
E = 320000
D = 128

NC = 2
NS = 16
NW = NC * NS

H = 2
E2 = E // H
CK = 200
NCHUNK = E2 // CK
NT = NCHUNK // NW
NP = 10112
ROWS_PER = NP // NS
ZR = 8
NPD = 10240
SPD = NPD // NS


def _gelu(x):
    return 0.5 * x * (1.0 + lax.erf(x * 0.7071067811865476))


def _worker_start():
    w = lax.axis_index("c") * NS + lax.axis_index("s")
    return w * NT


def _make_gather(hoff):
    def body(table, srcidx, dstidx, out, deg0, deg1,
             sh_deg, idx_all, didx_a, didx_b, rows_all, ones_v, iota_v,
             zeros_v, buf_v, sem_g, sem_wa, sem_wb, sem_da, sem_db):
        cid = lax.axis_index("c")
        sid = lax.axis_index("s")
        start = _worker_start()
        base = start * CK

        zero16 = jnp.zeros((16,), jnp.float32)
        one16 = jnp.ones((16,), jnp.float32)

        def fill_ones(i, carry):
            ones_v[pl.ds(i * 16, 16)] = one16
            return carry

        lax.fori_loop(0, CK // 16, fill_ones, 0)

        def fill_iota(i, carry):
            zeros_v[pl.ds(i * 16, 16)] = zero16
            iota_v[pl.ds(i * 16, 16)] = (lax.iota(jnp.int32, 16)
                                         + (sid * SPD + i * 16))
            return carry

        lax.fori_loop(0, SPD // 16, fill_iota, 0)

        pltpu.sync_copy(zeros_v, sh_deg.at[iota_v])
        plsc.subcore_barrier()

        pltpu.sync_copy(srcidx.at[pl.ds(hoff + base, NT * CK)], idx_all)

        didx = (didx_a, didx_b)
        dsem = (sem_da, sem_db)
        wsem = (sem_wa, sem_wb)

        def emit(t, b, first, last):
            loff = base + t * CK
            if not first:
                pltpu.make_async_copy(
                    rows_all.at[b],
                    out.at[pl.ds(loff - 2 * CK, CK)], wsem[b]).wait()
            pltpu.async_copy(table.at[idx_all.at[pl.ds(t * CK, CK)]],
                             rows_all.at[b], sem_g).wait()
            pltpu.async_copy(rows_all.at[b], out.at[pl.ds(loff, CK)], wsem[b])
            pltpu.make_async_copy(
                dstidx.at[pl.ds(hoff + loff, CK)], didx[b], dsem[b]).wait()
            if not last:
                pltpu.async_copy(dstidx.at[pl.ds(hoff + loff + CK, CK)],
                                 didx[1 - b], dsem[1 - b])
            pltpu.sync_copy(ones_v, sh_deg.at[didx[b]], add=True)

        pltpu.async_copy(dstidx.at[pl.ds(hoff + base, CK)], didx_a, sem_da)

        emit(0, 0, True, False)
        emit(1, 1, True, False)

        def pair(t2, carry):
            emit(2 * t2, 0, False, False)
            emit(2 * t2 + 1, 1, False, False)
            return carry

        lax.fori_loop(1, NT // 2, pair, 0)
        emit(NT - 1, 0, False, True)

        pltpu.make_async_copy(
            rows_all.at[1], out.at[pl.ds(base + (NT - 2) * CK, CK)],
            sem_wb).wait()
        pltpu.make_async_copy(
            rows_all.at[0], out.at[pl.ds(base + (NT - 1) * CK, CK)],
            sem_wa).wait()
        plsc.subcore_barrier()

        pltpu.sync_copy(sh_deg.at[iota_v], buf_v)

        @pl.when(cid == 0)
        def _():
            pltpu.sync_copy(buf_v, deg0.at[pl.ds(sid * SPD, SPD)])

        @pl.when(cid == 1)
        def _():
            pltpu.sync_copy(buf_v, deg1.at[pl.ds(sid * SPD, SPD)])

    return body


def _sc_gather_deg(node_embeddings, src, dst, hoff):
    mesh = plsc.VectorSubcoreMesh(core_axis_name="c", subcore_axis_name="s")
    f = pl.kernel(
        _make_gather(hoff),
        out_type=(
            jax.ShapeDtypeStruct((E2, D), jnp.float32),
            jax.ShapeDtypeStruct((NPD,), jnp.float32),
            jax.ShapeDtypeStruct((NPD,), jnp.float32),
        ),
        mesh=mesh,
        scratch_types=[
            pltpu.VMEM_SHARED((NPD,), jnp.float32),
            pltpu.VMEM((NT * CK,), jnp.int32),
            pltpu.VMEM((CK,), jnp.int32),
            pltpu.VMEM((CK,), jnp.int32),
            pltpu.VMEM((2, CK, D), jnp.float32),
            pltpu.VMEM((CK,), jnp.float32),
            pltpu.VMEM((SPD,), jnp.int32),
            pltpu.VMEM((SPD,), jnp.float32),
            pltpu.VMEM((SPD,), jnp.float32),
            pltpu.SemaphoreType.DMA,
            pltpu.SemaphoreType.DMA,
            pltpu.SemaphoreType.DMA,
            pltpu.SemaphoreType.DMA,
            pltpu.SemaphoreType.DMA,
        ],
    )
    return f(node_embeddings, src, dst)


def _make_scatter(hoff):
    def body(msg, dstidx, agg0, agg1, sh_agg,
             idx_a, idx_b, rows_v, zb,
             sem_ia, sem_ib):
        cid = lax.axis_index("c")
        sid = lax.axis_index("s")
        base = _worker_start() * CK

        zero16 = jnp.zeros((16,), jnp.float32)

        def fill_zb(i, carry):
            def fill_lane(j, c2):
                zb[i, pl.ds(j * 16, 16)] = zero16
                return c2
            return lax.fori_loop(0, D // 16, fill_lane, carry)

        lax.fori_loop(0, ZR, fill_zb, 0)

        def zero_shared(r, carry):
            pltpu.sync_copy(zb, sh_agg.at[pl.ds(sid * ROWS_PER + r * ZR, ZR)])
            return carry

        lax.fori_loop(0, ROWS_PER // ZR, zero_shared, 0)
        plsc.subcore_barrier()

        idx = (idx_a, idx_b)
        isem = (sem_ia, sem_ib)

        def emit(t, b, last):
            loff = base + t * CK
            pltpu.make_async_copy(dstidx.at[pl.ds(hoff + loff, CK)],
                                  idx[b], isem[b]).wait()
            if not last:
                pltpu.async_copy(dstidx.at[pl.ds(hoff + loff + CK, CK)],
                                 idx[1 - b], isem[1 - b])
            pltpu.sync_copy(msg.at[pl.ds(loff, CK)], rows_v)
            pltpu.sync_copy(rows_v, sh_agg.at[idx[b]], add=True)

        pltpu.async_copy(dstidx.at[pl.ds(hoff + base, CK)], idx_a, sem_ia)
        emit(0, 0, False)

        def pair(t2, carry):
            emit(2 * t2 + 1, 1, False)
            emit(2 * t2 + 2, 0, False)
            return carry

        lax.fori_loop(0, (NT - 3) // 2, pair, 0)
        emit(NT - 2, 1, False)
        emit(NT - 1, 0, True)
        plsc.subcore_barrier()

        row = sid * ROWS_PER

        @pl.when(cid == 0)
        def _():
            pltpu.sync_copy(sh_agg.at[pl.ds(row, ROWS_PER)],
                            agg0.at[pl.ds(row, ROWS_PER)])

        @pl.when(cid == 1)
        def _():
            pltpu.sync_copy(sh_agg.at[pl.ds(row, ROWS_PER)],
                            agg1.at[pl.ds(row, ROWS_PER)])

    return body


def _sc_scatter(messages, dst, hoff):
    mesh = plsc.VectorSubcoreMesh(core_axis_name="c", subcore_axis_name="s")
    f = pl.kernel(
        _make_scatter(hoff),
        out_type=(
            jax.ShapeDtypeStruct((NP, D), jnp.float32),
            jax.ShapeDtypeStruct((NP, D), jnp.float32),
        ),
        mesh=mesh,
        scratch_types=[
            pltpu.VMEM_SHARED((NP, D), jnp.float32),
            pltpu.VMEM((CK,), jnp.int32),
            pltpu.VMEM((CK,), jnp.int32),
            pltpu.VMEM((CK, D), jnp.float32),
            pltpu.VMEM((ZR, D), jnp.float32),
            pltpu.SemaphoreType.DMA,
            pltpu.SemaphoreType.DMA,
        ],
    )
    return f(messages, dst)


BE = 2000


def _edge_body(g_ref, e_ref, w1_ref, b1_ref, w2_ref, b2_ref, o_ref):
    x = g_ref[...] + e_ref[...]
    h = jnp.dot(x, w1_ref[...], preferred_element_type=jnp.float32) + b1_ref[...]
    h = _gelu(h)
    o_ref[...] = (jnp.dot(h, w2_ref[...], preferred_element_type=jnp.float32)
                  + b2_ref[...])


def _tc_edge_mlp(gathered, edge_embeddings, W1, b1, W2, b2, eblk):
    grid = (E2 // BE,)
    return pl.pallas_call(
        _edge_body,
        grid=grid,
        in_specs=[
            pl.BlockSpec((BE, D), lambda i: (i, 0)),
            pl.BlockSpec((BE, D), lambda i, _o=eblk: (i + _o, 0)),
            pl.BlockSpec((D, D), lambda i: (0, 0)),
            pl.BlockSpec((1, D), lambda i: (0, 0)),
            pl.BlockSpec((D, D), lambda i: (0, 0)),
            pl.BlockSpec((1, D), lambda i: (0, 0)),
        ],
        out_specs=pl.BlockSpec((BE, D), lambda i: (i, 0)),
        out_shape=jax.ShapeDtypeStruct((E2, D), jnp.float32),
    )(gathered, edge_embeddings, W1, b1.reshape(1, D), W2, b2.reshape(1, D))


BN = 2000


def _node_body(node_ref, a0_ref, a1_ref, a2_ref, a3_ref,
               d0_ref, d1_ref, d2_ref, d3_ref,
               w3_ref, b3_ref, w4_ref, b4_ref, g_ref, be_ref, o_ref):
    p = (a0_ref[...] + a1_ref[...]) + (a2_ref[...] + a3_ref[...])
    deg = (d0_ref[...] + d1_ref[...]) + (d2_ref[...] + d3_ref[...])
    a = p / jnp.maximum(deg, 1.0)
    t = _gelu(jnp.dot(a, w3_ref[...], preferred_element_type=jnp.float32)
              + b3_ref[...])
    u = jnp.dot(t, w4_ref[...], preferred_element_type=jnp.float32) + b4_ref[...]
    y = node_ref[...] + u
    mean = jnp.mean(y, axis=-1, keepdims=True)
    var = jnp.mean((y - mean) * (y - mean), axis=-1, keepdims=True)
    o_ref[...] = ((y - mean) * lax.rsqrt(var + 1e-5)) * g_ref[...] + be_ref[...]


def _tc_node_mlp(node_embeddings, aggs, degs, W3, b3, W4, b4, gamma, beta):
    grid = (N // BN,)
    agg_spec = pl.BlockSpec((BN, D), lambda i: (i, 0))
    deg_spec = pl.BlockSpec((BN, 1), lambda i: (i, 0))
    return pl.pallas_call(
        _node_body,
        grid=grid,
        in_specs=[
            pl.BlockSpec((BN, D), lambda i: (i, 0)),
            agg_spec, agg_spec, agg_spec, agg_spec,
            deg_spec, deg_spec, deg_spec, deg_spec,
            pl.BlockSpec((D, 2 * D), lambda i: (0, 0)),
            pl.BlockSpec((1, 2 * D), lambda i: (0, 0)),
            pl.BlockSpec((2 * D, D), lambda i: (0, 0)),
            pl.BlockSpec((1, D), lambda i: (0, 0)),
            pl.BlockSpec((1, D), lambda i: (0, 0)),
            pl.BlockSpec((1, D), lambda i: (0, 0)),
        ],
        out_specs=pl.BlockSpec((BN, D), lambda i: (i, 0)),
        out_shape=jax.ShapeDtypeStruct((N, D), jnp.float32),
    )(node_embeddings, *aggs, *degs,
      W3, b3.reshape(1, 2 * D), W4, b4.reshape(1, D),
      gamma.reshape(1, D), beta.reshape(1, D))


def kernel(node_embeddings, edge_index, edge_embeddings,
           W1, b1, W2, b2, W3, b3, W4, b4, gamma, beta):
    src = edge_index[0]
    dst = edge_index[1]

    g0, deg00, deg01 = _sc_gather_deg(node_embeddings, src, dst, 0)
    g1, deg10, deg11 = _sc_gather_deg(node_embeddings, src, dst, E2)

    m0 = _tc_edge_mlp(g0, edge_embeddings, W1, b1, W2, b2, 0)
    m1 = _tc_edge_mlp(g1, edge_embeddings, W1, b1, W2, b2, E2 // BE)

    a00, a01 = _sc_scatter(m0, dst, 0)
    a10, a11 = _sc_scatter(m1, dst, E2)

    degs = [d[:N].reshape(N, 1) for d in (deg00, deg01, deg10, deg11)]
    return _tc_node_mlp(node_embeddings, (a00, a01, a10, a11), degs,
                        W3, b3, W4, b4, gamma, beta)
